# Initial kernel scaffold; baseline (speedup 1.0000x reference)
import jax
import jax.numpy as jnp
from jax.experimental import pallas as pl


def kernel(distance, eps_ct_ij, lam_ct_ij, r_star_ij, edge_batch, num_graphs):
    raise NotImplementedError("write your pallas kernel here")


# speedup vs baseline: 11.5849x; 11.5849x over previous
import functools

import jax
import jax.numpy as jnp
from jax import lax
from jax.experimental import pallas as pl
from jax.experimental.pallas import tpu as pltpu
from jax.experimental.pallas import tpu_sc as plsc

E = 6_400_000
G = 4096
NC = 2
NS = 16
NW = NC * NS
EPW = E // NW
CHUNK = 8_000
NCHUNK = EPW // CHUNK
L = 16
VPC = CHUNK // L


def _sc_partials(distance, eps_ct, lam_ct, r_star, edge_batch):
    mesh = plsc.VectorSubcoreMesh(core_axis_name="c", subcore_axis_name="s")

    @functools.partial(
        pl.kernel,
        mesh=mesh,
        out_type=jax.ShapeDtypeStruct((NW, G), jnp.float32),
        compiler_params=pltpu.CompilerParams(needs_layout_passes=False),
        scratch_types=[
            pltpu.VMEM((CHUNK,), jnp.float32),
            pltpu.VMEM((CHUNK,), jnp.float32),
            pltpu.VMEM((CHUNK,), jnp.float32),
            pltpu.VMEM((CHUNK,), jnp.float32),
            pltpu.VMEM((CHUNK,), jnp.int32),
            pltpu.VMEM((G,), jnp.float32),
        ],
    )
    def k(d_hbm, ep_hbm, lm_hbm, rs_hbm, b_hbm, out_hbm,
          d_v, ep_v, lm_v, rs_v, b_v, acc):
        cid = lax.axis_index("c")
        sid = lax.axis_index("s")
        wid = cid * NS + sid

        zero = jnp.zeros((L,), jnp.float32)

        def zbody(i, carry):
            acc[pl.ds(i * L, L)] = zero
            return carry

        lax.fori_loop(0, G // L, zbody, 0)

        base = wid * EPW

        def chunk_body(ci, carry):
            start = base + ci * CHUNK
            pltpu.sync_copy(d_hbm.at[pl.ds(start, CHUNK)], d_v)
            pltpu.sync_copy(ep_hbm.at[pl.ds(start, CHUNK)], ep_v)
            pltpu.sync_copy(lm_hbm.at[pl.ds(start, CHUNK)], lm_v)
            pltpu.sync_copy(rs_hbm.at[pl.ds(start, CHUNK)], rs_v)
            pltpu.sync_copy(b_hbm.at[pl.ds(start, CHUNK)], b_v)

            def vbody(vi, inner):
                o = vi * L
                d = d_v[pl.ds(o, L)]
                ep = ep_v[pl.ds(o, L)]
                lm = lm_v[pl.ds(o, L)]
                rs = rs_v[pl.ds(o, L)]
                ib = b_v[pl.ds(o, L)]
                r = jnp.maximum(d, 1e-6)
                ratio = lm * jnp.maximum(rs, 1e-6) / r
                r2 = r * r
                pe = ep / (r2 * r2) * jnp.exp(-(ratio * ratio * ratio)) * 0.5
                plsc.addupdate_scatter(acc, [ib], pe)
                return inner

            lax.fori_loop(0, VPC, vbody, 0)
            return carry

        lax.fori_loop(0, NCHUNK, chunk_body, 0)
        pltpu.sync_copy(acc, out_hbm.at[wid])

    return k(distance, eps_ct, lam_ct, r_star, edge_batch)


def _tc_reduce(partials):
    def body(p_ref, o_ref):
        o_ref[...] = jnp.sum(p_ref[...], axis=0, keepdims=True)

    out = pl.pallas_call(
        body,
        out_shape=jax.ShapeDtypeStruct((1, G), jnp.float32),
    )(partials)
    return out.reshape(G)


def kernel(distance, eps_ct_ij, lam_ct_ij, r_star_ij, edge_batch, num_graphs):
    del num_graphs
    partials = _sc_partials(distance, eps_ct_ij, lam_ct_ij, r_star_ij,
                            edge_batch)
    return _tc_reduce(partials)


# speedup vs baseline: 16.3887x; 1.4147x over previous
import functools

import jax
import jax.numpy as jnp
from jax import lax
from jax.experimental import pallas as pl
from jax.experimental.pallas import tpu as pltpu
from jax.experimental.pallas import tpu_sc as plsc

E = 6_400_000
G = 4096
NC = 2
NS = 16
NW = NC * NS
EPW = E // NW
CHUNK = 4_000
NCHUNK = EPW // CHUNK
L = 16
VPC = CHUNK // L
ACC = L * G


def _sc_partials(distance, eps_ct, lam_ct, r_star, edge_batch):
    mesh = plsc.VectorSubcoreMesh(core_axis_name="c", subcore_axis_name="s")

    @functools.partial(
        pl.kernel,
        mesh=mesh,
        out_type=jax.ShapeDtypeStruct((NW, ACC), jnp.float32),
        compiler_params=pltpu.CompilerParams(needs_layout_passes=False),
        scratch_types=[
            pltpu.VMEM((CHUNK,), jnp.float32),
            pltpu.VMEM((CHUNK,), jnp.float32),
            pltpu.VMEM((CHUNK,), jnp.float32),
            pltpu.VMEM((CHUNK,), jnp.float32),
            pltpu.VMEM((CHUNK,), jnp.int32),
            pltpu.VMEM((CHUNK,), jnp.float32),
            pltpu.VMEM((CHUNK,), jnp.float32),
            pltpu.VMEM((CHUNK,), jnp.float32),
            pltpu.VMEM((CHUNK,), jnp.float32),
            pltpu.VMEM((CHUNK,), jnp.int32),
            pltpu.VMEM((ACC,), jnp.float32),
            pltpu.SemaphoreType.DMA,
            pltpu.SemaphoreType.DMA,
        ],
    )
    def k(d_hbm, ep_hbm, lm_hbm, rs_hbm, b_hbm, out_hbm,
          d_v0, ep_v0, lm_v0, rs_v0, b_v0,
          d_v1, ep_v1, lm_v1, rs_v1, b_v1, acc, sem0, sem1):
        bufs = ((d_v0, ep_v0, lm_v0, rs_v0, b_v0),
                (d_v1, ep_v1, lm_v1, rs_v1, b_v1))
        hbms = (d_hbm, ep_hbm, lm_hbm, rs_hbm, b_hbm)
        sems = (sem0, sem1)
        cid = lax.axis_index("c")
        sid = lax.axis_index("s")
        wid = cid * NS + sid

        zero = jnp.zeros((L,), jnp.float32)

        def zbody(i, carry):
            acc[pl.ds(i * L, L)] = zero
            return carry

        lax.fori_loop(0, ACC // L, zbody, 0)

        base = wid * EPW
        lane_off = lax.iota(jnp.int32, 16) * G

        def issue(ci, slot):
            start = base + ci * CHUNK
            for hbm, buf in zip(hbms, bufs[slot]):
                pltpu.async_copy(hbm.at[pl.ds(start, CHUNK)], buf, sems[slot])

        def drain(slot):
            src = pl.ds(0, CHUNK)
            for hbm, buf in zip(hbms, bufs[slot]):
                pltpu.make_async_copy(hbm.at[src], buf, sems[slot]).wait()

        def do_vec(slot, o):
            d_v, ep_v, lm_v, rs_v, b_v = bufs[slot]
            d = d_v[pl.ds(o, L)]
            ep = ep_v[pl.ds(o, L)]
            lm = lm_v[pl.ds(o, L)]
            rs = rs_v[pl.ds(o, L)]
            ib = b_v[pl.ds(o, L)]
            r = jnp.maximum(d, 1e-6)
            t = 1.0 / r
            ratio = lm * jnp.maximum(rs, 1e-6) * t
            t2 = t * t
            r3 = ratio * ratio * ratio
            pe = (0.5 * ep) * (t2 * t2) * jnp.exp(-r3)
            plsc.addupdate_scatter(acc, [ib + lane_off], pe)

        def compute(slot):
            def vbody(vi, carry):
                o = vi * (2 * L)
                do_vec(slot, o)
                do_vec(slot, o + L)
                return carry

            lax.fori_loop(0, VPC // 2, vbody, 0)

        issue(0, 0)

        def pair_body(t2i, carry):
            ci = t2i * 2
            issue(ci + 1, 1)
            drain(0)
            compute(0)

            @pl.when(ci + 2 < NCHUNK)
            def _():
                issue(ci + 2, 0)

            drain(1)
            compute(1)
            return carry

        lax.fori_loop(0, NCHUNK // 2, pair_body, 0)
        pltpu.sync_copy(acc, out_hbm.at[wid])

    return k(distance, eps_ct, lam_ct, r_star, edge_batch)


def _tc_reduce(partials):
    def body(p_ref, o_ref):
        o_ref[...] = jnp.sum(p_ref[...], axis=0, keepdims=True)

    out = pl.pallas_call(
        body,
        out_shape=jax.ShapeDtypeStruct((1, G), jnp.float32),
    )(partials)
    return out.reshape(G)


def kernel(distance, eps_ct_ij, lam_ct_ij, r_star_ij, edge_batch, num_graphs):
    del num_graphs
    partials = _sc_partials(distance, eps_ct_ij, lam_ct_ij, r_star_ij,
                            edge_batch)
    return _tc_reduce(partials.reshape(NW * L, G))


# speedup vs baseline: 36.8852x; 2.2507x over previous
import functools

import jax
import jax.numpy as jnp
from jax import lax
from jax.experimental import pallas as pl
from jax.experimental.pallas import tpu as pltpu
from jax.experimental.pallas import tpu_sc as plsc

E = 6_400_000
G = 4096
NC = 2
NS = 16
NW = NC * NS
EPW = E // NW
CHUNK = 4_000
NCHUNK = EPW // CHUNK
L = 16
VPC = CHUNK // L
ACC = L * G


def _sc_partials(distance, eps_ct, lam_ct, r_star, edge_batch):
    mesh = plsc.VectorSubcoreMesh(core_axis_name="c", subcore_axis_name="s")

    @functools.partial(
        pl.kernel,
        mesh=mesh,
        out_type=jax.ShapeDtypeStruct((NW, ACC), jnp.float32),
        compiler_params=pltpu.CompilerParams(needs_layout_passes=False),
        scratch_types=[
            pltpu.VMEM((CHUNK,), jnp.float32),
            pltpu.VMEM((CHUNK,), jnp.float32),
            pltpu.VMEM((CHUNK,), jnp.float32),
            pltpu.VMEM((CHUNK,), jnp.float32),
            pltpu.VMEM((CHUNK,), jnp.int32),
            pltpu.VMEM((CHUNK,), jnp.float32),
            pltpu.VMEM((CHUNK,), jnp.float32),
            pltpu.VMEM((CHUNK,), jnp.float32),
            pltpu.VMEM((CHUNK,), jnp.float32),
            pltpu.VMEM((CHUNK,), jnp.int32),
            pltpu.VMEM((ACC,), jnp.float32),
            pltpu.SemaphoreType.DMA,
            pltpu.SemaphoreType.DMA,
        ],
    )
    def k(d_hbm, ep_hbm, lm_hbm, rs_hbm, b_hbm, out_hbm,
          d_v0, ep_v0, lm_v0, rs_v0, b_v0,
          d_v1, ep_v1, lm_v1, rs_v1, b_v1, acc, sem0, sem1):
        bufs = ((d_v0, ep_v0, lm_v0, rs_v0, b_v0),
                (d_v1, ep_v1, lm_v1, rs_v1, b_v1))
        hbms = (d_hbm, ep_hbm, lm_hbm, rs_hbm, b_hbm)
        sems = (sem0, sem1)
        cid = lax.axis_index("c")
        sid = lax.axis_index("s")
        wid = cid * NS + sid

        zero = jnp.zeros((L,), jnp.float32)

        @plsc.parallel_loop(0, ACC // L, 1, unroll=8)
        def _(i):
            acc[pl.ds(i * L, L)] = zero

        base = wid * EPW
        lane_off = lax.iota(jnp.int32, 16) * G

        def issue(ci, slot):
            start = base + ci * CHUNK
            for hbm, buf in zip(hbms, bufs[slot]):
                pltpu.async_copy(hbm.at[pl.ds(start, CHUNK)], buf, sems[slot])

        def drain(slot):
            src = pl.ds(0, CHUNK)
            for hbm, buf in zip(hbms, bufs[slot]):
                pltpu.make_async_copy(hbm.at[src], buf, sems[slot]).wait()

        def do_vec(slot, o):
            d_v, ep_v, lm_v, rs_v, b_v = bufs[slot]
            d = d_v[pl.ds(o, L)]
            ep = ep_v[pl.ds(o, L)]
            lm = lm_v[pl.ds(o, L)]
            rs = rs_v[pl.ds(o, L)]
            ib = b_v[pl.ds(o, L)]
            r = jnp.maximum(d, 1e-6)
            t = 1.0 / r
            ratio = lm * jnp.maximum(rs, 1e-6) * t
            t2 = t * t
            r3 = ratio * ratio * ratio
            pe = (0.5 * ep) * (t2 * t2) * jnp.exp(-r3)
            plsc.addupdate_scatter(acc, [ib + lane_off], pe)

        def compute(slot):
            @plsc.parallel_loop(0, VPC, 1, unroll=8)
            def _(vi):
                do_vec(slot, vi * L)

        issue(0, 0)

        def pair_body(t2i, carry):
            ci = t2i * 2
            issue(ci + 1, 1)
            drain(0)
            compute(0)

            @pl.when(ci + 2 < NCHUNK)
            def _():
                issue(ci + 2, 0)

            drain(1)
            compute(1)
            return carry

        lax.fori_loop(0, NCHUNK // 2, pair_body, 0)
        pltpu.sync_copy(acc, out_hbm.at[wid])

    return k(distance, eps_ct, lam_ct, r_star, edge_batch)


def _tc_reduce(partials):
    def body(p_ref, o_ref):
        o_ref[...] = jnp.sum(p_ref[...], axis=0, keepdims=True)

    out = pl.pallas_call(
        body,
        out_shape=jax.ShapeDtypeStruct((1, G), jnp.float32),
    )(partials)
    return out.reshape(G)


def kernel(distance, eps_ct_ij, lam_ct_ij, r_star_ij, edge_batch, num_graphs):
    del num_graphs
    partials = _sc_partials(distance, eps_ct_ij, lam_ct_ij, r_star_ij,
                            edge_batch)
    return _tc_reduce(partials.reshape(NW * L, G))
